# baseline (speedup 1.0000x reference)
import jax
import jax.numpy as jnp
from jax.experimental import pallas as pl
from jax.experimental.pallas import tpu as pltpu

_G = 64
_BN = 25000
_LOG2E = 1.4426950408889634


def _pool_kernel(seg_ref, x_ref, wg_ref, wf_ref, bf_ref, out_ref,
                 b_ref, s_ref, acc_ref):
    i = pl.program_id(0)
    nsteps = pl.num_programs(0)

    @pl.when(i == 0)
    def _init():
        b_ref[...] = jnp.full_like(b_ref, -1e30)
        s_ref[...] = jnp.zeros_like(s_ref)
        acc_ref[...] = jnp.zeros_like(acc_ref)

    x = x_ref[...]
    seg = seg_ref[0]
    bn = x.shape[0]

    g = jax.lax.dot_general(wg_ref[...], x, (((1,), (1,)), ((), ())),
                            preferred_element_type=jnp.float32)

    b_old = b_ref[0, 0]
    b_new = jnp.maximum(b_old, jnp.max(g))
    scale = jnp.exp2(b_old - b_new)

    eg = jnp.exp2(g - b_new)

    onehot = seg == jax.lax.broadcasted_iota(jnp.int32, (_G, bn), 0)
    p = jnp.where(onehot, eg, 0.0)

    ones = jnp.ones((bn, 1), dtype=jnp.float32)
    sblk = jax.lax.dot_general(p, ones, (((1,), (0,)), ((), ())),
                               preferred_element_type=jnp.float32)
    s_ref[...] = s_ref[...] * scale + sblk
    contrib = jax.lax.dot_general(p, x, (((1,), (0,)), ((), ())),
                                  preferred_element_type=jnp.float32)
    acc_ref[...] = acc_ref[...] * scale + contrib
    b_ref[...] = jnp.full_like(b_ref, b_new)

    @pl.when(i == nsteps - 1)
    def _epilogue():
        s = s_ref[...]
        inv = jnp.where(s > 0, 1.0 / s, 0.0)
        pooled = acc_ref[...] * inv
        ro = jnp.dot(pooled, wf_ref[...],
                     preferred_element_type=jnp.float32)
        ind = jnp.where(s > 0, 1.0, 0.0)
        out_ref[...] = ro + ind * bf_ref[...]


def kernel(feat, segment_ids, W_gate, W_feat, b_feat):
    n, d = feat.shape
    h = W_feat.shape[1]
    nb = n // _BN
    seg3d = segment_ids.reshape(nb, 1, _BN)
    wg_row = W_gate.reshape(1, d) * jnp.float32(_LOG2E)
    bf2 = b_feat.reshape(1, h)
    return pl.pallas_call(
        _pool_kernel,
        grid=(nb,),
        in_specs=[
            pl.BlockSpec((1, 1, _BN), lambda i: (i, 0, 0)),
            pl.BlockSpec((_BN, d), lambda i: (i, 0)),
            pl.BlockSpec((1, d), lambda i: (0, 0)),
            pl.BlockSpec((d, h), lambda i: (0, 0)),
            pl.BlockSpec((1, h), lambda i: (0, 0)),
        ],
        out_specs=pl.BlockSpec((_G, h), lambda i: (0, 0)),
        out_shape=jax.ShapeDtypeStruct((_G, h), jnp.float32),
        scratch_shapes=[
            pltpu.VMEM((1, 1), jnp.float32),
            pltpu.VMEM((_G, 1), jnp.float32),
            pltpu.VMEM((_G, d), jnp.float32),
        ],
        compiler_params=pltpu.CompilerParams(
            dimension_semantics=("arbitrary",)),
    )(seg3d, feat, wg_row, W_feat, bf2)


# speedup vs baseline: 1.0565x; 1.0565x over previous
import jax
import jax.numpy as jnp
from jax.experimental import pallas as pl
from jax.experimental.pallas import tpu as pltpu

_G = 64
_BN = 25000
_LOG2E = 1.4426950408889634


def _pool_kernel(seg_ref, x_ref, wg_ref, wf_ref, bf_ref, out_ref,
                 b_ref, s_ref, acc_ref):
    i = pl.program_id(0)
    nsteps = pl.num_programs(0)

    @pl.when(i == 0)
    def _init():
        b_ref[...] = jnp.full_like(b_ref, -1e30)
        s_ref[...] = jnp.zeros_like(s_ref)
        acc_ref[...] = jnp.zeros_like(acc_ref)

    x = x_ref[...]
    seg = seg_ref[0]
    bn = x.shape[0]

    wgl = wg_ref[...] * jnp.float32(_LOG2E)
    g = jax.lax.dot_general(wgl, x, (((1,), (1,)), ((), ())),
                            preferred_element_type=jnp.float32)

    b_old = b_ref[0, 0]
    b_new = jnp.maximum(b_old, jnp.max(g))
    scale = jnp.exp2(b_old - b_new)

    eg = jnp.exp2(g - b_new)

    onehot = seg == jax.lax.broadcasted_iota(jnp.int32, (_G, bn), 0)
    p = jnp.where(onehot, eg, 0.0)

    ones = jnp.ones((bn, 1), dtype=jnp.float32)
    sblk = jax.lax.dot_general(p, ones, (((1,), (0,)), ((), ())),
                               preferred_element_type=jnp.float32)
    s_ref[...] = s_ref[...] * scale + sblk
    contrib = jax.lax.dot_general(p, x, (((1,), (0,)), ((), ())),
                                  preferred_element_type=jnp.float32)
    acc_ref[...] = acc_ref[...] * scale + contrib
    b_ref[...] = jnp.full_like(b_ref, b_new)

    @pl.when(i == nsteps - 1)
    def _epilogue():
        s = s_ref[...]
        inv = jnp.where(s > 0, 1.0 / s, 0.0)
        pooled = acc_ref[...] * inv
        ro = jnp.dot(pooled, wf_ref[...],
                     preferred_element_type=jnp.float32)
        ind = jnp.where(s > 0, 1.0, 0.0)
        out_ref[...] = ro + ind * bf_ref[...]


def kernel(feat, segment_ids, W_gate, W_feat, b_feat):
    n, d = feat.shape
    h = W_feat.shape[1]
    nb = n // _BN
    seg3d = segment_ids.reshape(nb, 1, _BN)
    wg_row = W_gate.reshape(1, d)
    bf2 = b_feat.reshape(1, h)
    return pl.pallas_call(
        _pool_kernel,
        grid=(nb,),
        in_specs=[
            pl.BlockSpec((1, 1, _BN), lambda i: (i, 0, 0)),
            pl.BlockSpec((_BN, d), lambda i: (i, 0)),
            pl.BlockSpec((1, d), lambda i: (0, 0)),
            pl.BlockSpec((d, h), lambda i: (0, 0)),
            pl.BlockSpec((1, h), lambda i: (0, 0)),
        ],
        out_specs=pl.BlockSpec((_G, h), lambda i: (0, 0)),
        out_shape=jax.ShapeDtypeStruct((_G, h), jnp.float32),
        scratch_shapes=[
            pltpu.VMEM((1, 1), jnp.float32),
            pltpu.VMEM((_G, 1), jnp.float32),
            pltpu.VMEM((_G, d), jnp.float32),
        ],
        compiler_params=pltpu.CompilerParams(
            dimension_semantics=("arbitrary",)),
    )(seg3d, feat, wg_row, W_feat, bf2)
